# baseline (speedup 1.0000x reference)
import functools

import jax
import jax.numpy as jnp
from jax import lax
from jax.experimental import pallas as pl
from jax.experimental.pallas import tpu as pltpu
from jax.experimental.pallas import tpu_sc as plsc

NUM_ENTITIES = 1000000
NUM_RELATIONS = 1000
EMBED_DIM = 64
BATCH = 16384

_INFO = plsc.get_sparse_core_info()
_NC = _INFO.num_cores
_NS = _INFO.num_subcores
_L = _INFO.num_lanes
_NW = _NC * _NS
_BPW = BATCH // _NW
_CHUNK = 32
_NCHUNK = _BPW // _CHUNK


def _sqrt16(x):
    xc = jnp.maximum(x, jnp.float32(1e-35))
    i = plsc.bitcast(xc, jnp.int32)
    y = plsc.bitcast(jnp.int32(0x5F3759DF) - (i >> 1), jnp.float32)
    half = jnp.float32(0.5) * xc
    for _ in range(4):
        y = y * (jnp.float32(1.5) - half * y * y)
    return x * y


def _transe_body(head_hbm, rel_hbm, tail_hbm, ent2d, rel2, out_hbm,
                 hidx, ridx, tidx, sridx,
                 hslab0, tslab0, hslab1, tslab1, rrows0,
                 score_v, sem0):
    wid = lax.axis_index("s") * _NC + lax.axis_index("c")
    base = wid * _BPW

    pltpu.sync_copy(head_hbm.at[pl.ds(base, _BPW)], hidx)
    pltpu.sync_copy(rel_hbm.at[pl.ds(base, _BPW)], ridx)
    pltpu.sync_copy(tail_hbm.at[pl.ds(base, _BPW)], tidx)

    lane = lax.iota(jnp.int32, _L)
    zero = jnp.zeros((_L,), jnp.float32)

    for v in range(_BPW // _L):
        sl = pl.ds(v * _L, _L)
        sridx[sl] = ridx[sl] >> 1

    def chunk_body(c, _):
        hp = hidx[pl.ds(c * _CHUNK, _L)] >> 3
        tp = tidx[pl.ds(c * _CHUNK, _L)] >> 3
        hp2 = hidx[pl.ds(c * _CHUNK + _L, _L)] >> 3
        tp2 = tidx[pl.ds(c * _CHUNK + _L, _L)] >> 3
        pltpu.async_copy(
            rel2.at[sridx.at[pl.ds(c * _CHUNK, _CHUNK)]], rrows0, sem0)
        for s in range(_L):
            pltpu.async_copy(ent2d.at[hp[s]], hslab0.at[s], sem0)
            pltpu.async_copy(ent2d.at[tp[s]], tslab0.at[s], sem0)
            pltpu.async_copy(ent2d.at[hp2[s]], hslab1.at[s], sem0)
            pltpu.async_copy(ent2d.at[tp2[s]], tslab1.at[s], sem0)
        pltpu.make_async_copy(
            rel2.at[pl.ds(0, _CHUNK)], rrows0, sem0).wait()
        pltpu.make_async_copy(ent2d.at[pl.ds(0, _L)], hslab0, sem0).wait()
        pltpu.make_async_copy(ent2d.at[pl.ds(0, _L)], tslab0, sem0).wait()
        pltpu.make_async_copy(ent2d.at[pl.ds(0, _L)], hslab1, sem0).wait()
        pltpu.make_async_copy(ent2d.at[pl.ds(0, _L)], tslab1, sem0).wait()

        for g in range(2):
            gsl = pl.ds(c * _CHUNK + g * _L, _L)
            hslab = (hslab0, hslab1)[g]
            tslab = (tslab0, tslab1)[g]
            hrow = hidx[gsl] & 7
            trow = tidx[gsl] & 7
            roff = (ridx[gsl] & 1) * EMBED_DIM
            rlane = g * _L + lane
            acc = zero
            for k in range(EMBED_DIM):
                kv = jnp.full((_L,), k, jnp.int32)
                h = plsc.load_gather(hslab, [lane, hrow, kv])
                t = plsc.load_gather(tslab, [lane, trow, kv])
                r = plsc.load_gather(rrows0, [rlane, roff + kv])
                d = h + r - t
                acc = acc + d * d
            score_v[gsl] = _sqrt16(acc)
        return 0

    lax.fori_loop(0, _NCHUNK, chunk_body, 0)

    pltpu.sync_copy(score_v, out_hbm.at[pl.ds(base, _BPW)])


@jax.jit
def kernel(head, relation, tail, entity_emb, relation_emb):
    ent3 = entity_emb.reshape(NUM_ENTITIES // 8, 8, EMBED_DIM)
    rel2 = relation_emb.reshape(NUM_RELATIONS // 2, 2 * EMBED_DIM)
    mesh = plsc.VectorSubcoreMesh(core_axis_name="c", subcore_axis_name="s")
    k = functools.partial(
        pl.kernel,
        mesh=mesh,
        out_type=jax.ShapeDtypeStruct((BATCH,), jnp.float32),
        scratch_types=[
            pltpu.VMEM((_BPW,), jnp.int32),
            pltpu.VMEM((_BPW,), jnp.int32),
            pltpu.VMEM((_BPW,), jnp.int32),
            pltpu.VMEM((_BPW,), jnp.int32),
            pltpu.VMEM((_L, 8, EMBED_DIM), jnp.float32),
            pltpu.VMEM((_L, 8, EMBED_DIM), jnp.float32),
            pltpu.VMEM((_L, 8, EMBED_DIM), jnp.float32),
            pltpu.VMEM((_L, 8, EMBED_DIM), jnp.float32),
            pltpu.VMEM((_CHUNK, 2 * EMBED_DIM), jnp.float32),
            pltpu.VMEM((_BPW,), jnp.float32),
            pltpu.SemaphoreType.DMA,
        ],
        compiler_params=pltpu.CompilerParams(
            needs_layout_passes=False, use_tc_tiling_on_sc=True),
    )(_transe_body)
    return k(head, relation, tail, ent3, rel2)


# speedup vs baseline: 1.1316x; 1.1316x over previous
import functools

import jax
import jax.numpy as jnp
from jax import lax
from jax.experimental import pallas as pl
from jax.experimental.pallas import tpu as pltpu
from jax.experimental.pallas import tpu_sc as plsc

NUM_ENTITIES = 1000000
NUM_RELATIONS = 1000
EMBED_DIM = 64
BATCH = 16384

_INFO = plsc.get_sparse_core_info()
_NC = _INFO.num_cores
_NS = _INFO.num_subcores
_L = _INFO.num_lanes
_NW = _NC * _NS
_BPW = BATCH // _NW
_CHUNK = 32
_NCHUNK = _BPW // _CHUNK


def _sqrt16(x):
    xc = jnp.maximum(x, jnp.float32(1e-35))
    i = plsc.bitcast(xc, jnp.int32)
    y = plsc.bitcast(jnp.int32(0x5F3759DF) - (i >> 1), jnp.float32)
    half = jnp.float32(0.5) * xc
    for _ in range(4):
        y = y * (jnp.float32(1.5) - half * y * y)
    return x * y


def _transe_body(head_hbm, rel_hbm, tail_hbm, ent2d, rel2, out_hbm,
                 hidx, ridx, tidx, sridx,
                 hslab0, tslab0, hslab1, tslab1, rrows0,
                 score_v, sem0):
    wid = lax.axis_index("s") * _NC + lax.axis_index("c")
    base = wid * _BPW

    pltpu.sync_copy(head_hbm.at[pl.ds(base, _BPW)], hidx)
    pltpu.sync_copy(rel_hbm.at[pl.ds(base, _BPW)], ridx)
    pltpu.sync_copy(tail_hbm.at[pl.ds(base, _BPW)], tidx)

    lane = lax.iota(jnp.int32, _L)
    zero = jnp.zeros((_L,), jnp.float32)

    for v in range(_BPW // _L):
        sl = pl.ds(v * _L, _L)
        sridx[sl] = ridx[sl] >> 1

    def chunk_body(c, _):
        hp = hidx[pl.ds(c * _CHUNK, _L)] >> 3
        tp = tidx[pl.ds(c * _CHUNK, _L)] >> 3
        hp2 = hidx[pl.ds(c * _CHUNK + _L, _L)] >> 3
        tp2 = tidx[pl.ds(c * _CHUNK + _L, _L)] >> 3
        pltpu.async_copy(
            rel2.at[sridx.at[pl.ds(c * _CHUNK, _CHUNK)]], rrows0, sem0)
        for s in range(_L):
            pltpu.async_copy(ent2d.at[hp[s]], hslab0.at[s], sem0)
            pltpu.async_copy(ent2d.at[tp[s]], tslab0.at[s], sem0)
            pltpu.async_copy(ent2d.at[hp2[s]], hslab1.at[s], sem0)
            pltpu.async_copy(ent2d.at[tp2[s]], tslab1.at[s], sem0)
        pltpu.make_async_copy(
            rel2.at[pl.ds(0, _CHUNK)], rrows0, sem0).wait()
        pltpu.make_async_copy(ent2d.at[pl.ds(0, _L)], hslab0, sem0).wait()
        pltpu.make_async_copy(ent2d.at[pl.ds(0, _L)], tslab0, sem0).wait()
        pltpu.make_async_copy(ent2d.at[pl.ds(0, _L)], hslab1, sem0).wait()
        pltpu.make_async_copy(ent2d.at[pl.ds(0, _L)], tslab1, sem0).wait()

        for g in range(2):
            gsl = pl.ds(c * _CHUNK + g * _L, _L)
            hslab = (hslab0, hslab1)[g]
            tslab = (tslab0, tslab1)[g]
            hv = hidx[gsl]
            tv = tidx[gsl]
            rv = ridx[gsl]
            hrow = hv & 7
            trow = tv & 7
            roff = (rv & 1) * EMBED_DIM
            out_vec = zero
            for l in range(_L):
                hr = hrow[l]
                tr = trow[l]
                ro = roff[l]
                acc = zero
                for q in range(EMBED_DIM // _L):
                    h = hslab[l, hr, pl.ds(q * _L, _L)]
                    t = tslab[l, tr, pl.ds(q * _L, _L)]
                    r = rrows0[g * _L + l, pl.ds(ro + q * _L, _L)]
                    d = h + r - t
                    acc = acc + d * d
                s = jnp.sum(acc)
                out_vec = jnp.where(lane == l, s, out_vec)
            score_v[gsl] = _sqrt16(out_vec)
        return 0

    lax.fori_loop(0, _NCHUNK, chunk_body, 0)

    pltpu.sync_copy(score_v, out_hbm.at[pl.ds(base, _BPW)])


@jax.jit
def kernel(head, relation, tail, entity_emb, relation_emb):
    ent3 = entity_emb.reshape(NUM_ENTITIES // 8, 8, EMBED_DIM)
    rel2 = relation_emb.reshape(NUM_RELATIONS // 2, 2 * EMBED_DIM)
    mesh = plsc.VectorSubcoreMesh(core_axis_name="c", subcore_axis_name="s")
    k = functools.partial(
        pl.kernel,
        mesh=mesh,
        out_type=jax.ShapeDtypeStruct((BATCH,), jnp.float32),
        scratch_types=[
            pltpu.VMEM((_BPW,), jnp.int32),
            pltpu.VMEM((_BPW,), jnp.int32),
            pltpu.VMEM((_BPW,), jnp.int32),
            pltpu.VMEM((_BPW,), jnp.int32),
            pltpu.VMEM((_L, 8, EMBED_DIM), jnp.float32),
            pltpu.VMEM((_L, 8, EMBED_DIM), jnp.float32),
            pltpu.VMEM((_L, 8, EMBED_DIM), jnp.float32),
            pltpu.VMEM((_L, 8, EMBED_DIM), jnp.float32),
            pltpu.VMEM((_CHUNK, 2 * EMBED_DIM), jnp.float32),
            pltpu.VMEM((_BPW,), jnp.float32),
            pltpu.SemaphoreType.DMA,
        ],
        compiler_params=pltpu.CompilerParams(
            needs_layout_passes=False, use_tc_tiling_on_sc=True),
    )(_transe_body)
    return k(head, relation, tail, ent3, rel2)
